# baseline (speedup 1.0000x reference)
import functools
import math

import jax
import jax.numpy as jnp
from jax.experimental import pallas as pl

B = 2
N = 1000
C = 256
FH = 50
FW = 50
POOL = 7
SCALE = 1.0 / 16.0
NUM_CLASSES = 91
HID = 1024
IMG_H = 800.0
IMG_W = 800.0
SCORE_THRESH = 0.05
NMS_THRESH = 0.5
DET_PER_IMG = 100
PRE_NMS_TOPK = 2000
BBOX_XFORM_CLIP = math.log(1000.0 / 16.0)
NEG = -1e10

K_PAD = 2048


def _mm_body(x_ref, w_ref, b_ref, o_ref, *, nk, act):
    j = pl.program_id(1)

    @pl.when(j == 0)
    def _():
        o_ref[...] = jnp.zeros_like(o_ref)

    o_ref[...] += jnp.dot(x_ref[...], w_ref[...],
                          preferred_element_type=jnp.float32)

    @pl.when(j == nk - 1)
    def _():
        r = o_ref[...] + b_ref[...]
        if act:
            r = jnp.maximum(r, 0.0)
        o_ref[...] = r


def _pallas_mm(x, w, b, *, act, bm, bk):
    M, K = x.shape
    Nn = w.shape[1]
    nm = M // bm
    nk = K // bk
    return pl.pallas_call(
        functools.partial(_mm_body, nk=nk, act=act),
        grid=(nm, nk),
        in_specs=[
            pl.BlockSpec((bm, bk), lambda i, j: (i, j)),
            pl.BlockSpec((bk, Nn), lambda i, j: (j, 0)),
            pl.BlockSpec((1, Nn), lambda i, j: (0, 0)),
        ],
        out_specs=pl.BlockSpec((bm, Nn), lambda i, j: (i, 0)),
        out_shape=jax.ShapeDtypeStruct((M, Nn), jnp.float32),
    )(x, w, b.reshape(1, Nn))


def _nms_body(boxes_ref, v_ref, keep_ref):
    keep_ref[...] = v_ref[...]
    x1 = boxes_ref[0:1, :]
    y1 = boxes_ref[1:2, :]
    x2 = boxes_ref[2:3, :]
    y2 = boxes_ref[3:4, :]
    area = (x2 - x1) * (y2 - y1)
    idx = jax.lax.broadcasted_iota(jnp.int32, (1, K_PAD), 1)

    def body(i, _):
        sel = (idx == i)
        xi1 = jnp.sum(jnp.where(sel, x1, 0.0), keepdims=True)
        yi1 = jnp.sum(jnp.where(sel, y1, 0.0), keepdims=True)
        xi2 = jnp.sum(jnp.where(sel, x2, 0.0), keepdims=True)
        yi2 = jnp.sum(jnp.where(sel, y2, 0.0), keepdims=True)
        w = jnp.maximum(jnp.minimum(x2, xi2) - jnp.maximum(x1, xi1), 0.0)
        h = jnp.maximum(jnp.minimum(y2, yi2) - jnp.maximum(y1, yi1), 0.0)
        inter = w * h
        union = area + (xi2 - xi1) * (yi2 - yi1) - inter
        iou_row = inter / jnp.maximum(union, 1e-9)
        keep = keep_ref[...]
        keep_i = jnp.sum(jnp.where(sel, keep, 0.0), keepdims=True)
        sup = (iou_row > NMS_THRESH) & (idx > i) & (keep_i > 0.0)
        keep_ref[...] = jnp.where(sup, 0.0, keep)
        return 0

    jax.lax.fori_loop(0, PRE_NMS_TOPK, body, 0)


def _pallas_nms(boxes_t, v):
    return pl.pallas_call(
        _nms_body,
        out_shape=jax.ShapeDtypeStruct((1, K_PAD), jnp.float32),
    )(boxes_t, v)


def _bilinear(feat, ys, xs):
    Cc, H, W = feat.shape
    y0 = jnp.floor(ys)
    x0 = jnp.floor(xs)
    ly = ys - y0
    lx = xs - x0
    hy = 1.0 - ly
    hx = 1.0 - lx
    y0i = jnp.clip(y0.astype(jnp.int32), 0, H - 1)
    y1i = jnp.clip(y0.astype(jnp.int32) + 1, 0, H - 1)
    x0i = jnp.clip(x0.astype(jnp.int32), 0, W - 1)
    x1i = jnp.clip(x0.astype(jnp.int32) + 1, 0, W - 1)
    f = feat.reshape(Cc, H * W)
    v00 = f[:, y0i * W + x0i]
    v01 = f[:, y0i * W + x1i]
    v10 = f[:, y1i * W + x0i]
    v11 = f[:, y1i * W + x1i]
    return v00 * (hy * hx) + v01 * (hy * lx) + v10 * (ly * hx) + v11 * (ly * lx)


def _roi_align(feat, boxes):
    b = boxes * SCALE
    x1 = b[:, 0]
    y1 = b[:, 1]
    x2 = b[:, 2]
    y2 = b[:, 3]
    rw = jnp.maximum(x2 - x1, 1.0)
    rh = jnp.maximum(y2 - y1, 1.0)
    g = (jnp.arange(POOL, dtype=jnp.float32) + 0.5) / POOL
    xs = x1[:, None] + g[None, :] * rw[:, None]
    ys = y1[:, None] + g[None, :] * rh[:, None]
    n = boxes.shape[0]
    yy = jnp.broadcast_to(ys[:, :, None], (n, POOL, POOL)).reshape(-1)
    xx = jnp.broadcast_to(xs[:, None, :], (n, POOL, POOL)).reshape(-1)
    vals = _bilinear(feat, yy, xx)
    vals = vals.reshape(feat.shape[0], n, POOL, POOL)
    return jnp.transpose(vals, (1, 0, 2, 3))


def _decode(reg, props):
    w = props[:, 2] - props[:, 0]
    h = props[:, 3] - props[:, 1]
    cx = props[:, 0] + 0.5 * w
    cy = props[:, 1] + 0.5 * h
    reg = reg.reshape(reg.shape[0], NUM_CLASSES, 4)
    dx = reg[..., 0] / 10.0
    dy = reg[..., 1] / 10.0
    dw = jnp.minimum(reg[..., 2] / 5.0, BBOX_XFORM_CLIP)
    dh = jnp.minimum(reg[..., 3] / 5.0, BBOX_XFORM_CLIP)
    pcx = dx * w[:, None] + cx[:, None]
    pcy = dy * h[:, None] + cy[:, None]
    pw = jnp.exp(dw) * w[:, None]
    ph = jnp.exp(dh) * h[:, None]
    return jnp.stack([pcx - 0.5 * pw, pcy - 0.5 * ph,
                      pcx + 0.5 * pw, pcy + 0.5 * ph], axis=-1)


def _postprocess_one(boxes, scores):
    bx1 = jnp.clip(boxes[..., 0], 0.0, IMG_W)
    by1 = jnp.clip(boxes[..., 1], 0.0, IMG_H)
    bx2 = jnp.clip(boxes[..., 2], 0.0, IMG_W)
    by2 = jnp.clip(boxes[..., 3], 0.0, IMG_H)
    boxes = jnp.stack([bx1, by1, bx2, by2], axis=-1)
    boxes = boxes[:, 1:, :].reshape(-1, 4)
    scores = scores[:, 1:].reshape(-1)
    labels = jnp.tile(jnp.arange(1, NUM_CLASSES), boxes.shape[0] // (NUM_CLASSES - 1))
    ws = boxes[:, 2] - boxes[:, 0]
    hs = boxes[:, 3] - boxes[:, 1]
    valid = (scores > SCORE_THRESH) & (ws >= 0.01) & (hs >= 0.01)
    masked = jnp.where(valid, scores, NEG)
    _, top_i = jax.lax.top_k(masked, PRE_NMS_TOPK)
    b = boxes[top_i]
    s = scores[top_i]
    l = labels[top_i]
    v = valid[top_i]
    offs = l.astype(jnp.float32) * (jnp.maximum(IMG_W, IMG_H) + 1.0)
    bn = b + offs[:, None]

    boxes_t = jnp.zeros((4, K_PAD), jnp.float32).at[:, :PRE_NMS_TOPK].set(bn.T)
    v_f = jnp.zeros((1, K_PAD), jnp.float32).at[0, :PRE_NMS_TOPK].set(
        v.astype(jnp.float32))
    keep = _pallas_nms(boxes_t, v_f)[0, :PRE_NMS_TOPK] > 0.0

    final = jnp.where(keep, s, NEG)
    _, fi = jax.lax.top_k(final, DET_PER_IMG)
    fv = keep[fi]
    fb = jnp.where(fv[:, None], b[fi], 0.0)
    fs = jnp.where(fv, s[fi], 0.0)
    fl = jnp.where(fv, l[fi], 0).astype(jnp.float32)
    return jnp.concatenate([fb, fs[:, None], fl[:, None]], axis=1)


def kernel(features, proposals, W6, b6, W7, b7, Wc, bc, Wb, bb):
    pooled = jnp.concatenate(
        [_roi_align(features[i], proposals[i]) for i in range(B)], axis=0)
    x = pooled.reshape(pooled.shape[0], -1)
    M = B * N
    M_pad = 2048
    x = jnp.zeros((M_pad, x.shape[1]), jnp.float32).at[:M].set(x)

    h = _pallas_mm(x, W6, b6, act=True, bm=256, bk=1792)
    h = _pallas_mm(h, W7, b7, act=True, bm=256, bk=1024)

    wc_p = jnp.zeros((HID, 128), jnp.float32).at[:, :NUM_CLASSES].set(Wc)
    wb_p = jnp.zeros((HID, 384), jnp.float32).at[:, :NUM_CLASSES * 4].set(Wb)
    w_head = jnp.concatenate([wc_p, wb_p], axis=1)
    b_head = jnp.zeros((512,), jnp.float32)
    b_head = b_head.at[:NUM_CLASSES].set(bc).at[128:128 + NUM_CLASSES * 4].set(bb)
    head = _pallas_mm(h, w_head, b_head, act=False, bm=256, bk=1024)

    class_logits = head[:M, :NUM_CLASSES]
    box_reg = head[:M, 128:128 + NUM_CLASSES * 4]

    props = proposals.reshape(-1, 4)
    pred_boxes = _decode(box_reg, props)
    pred_scores = jax.nn.softmax(class_logits, axis=-1)
    dets = [_postprocess_one(pred_boxes[i * N:(i + 1) * N],
                             pred_scores[i * N:(i + 1) * N]) for i in range(B)]
    return jnp.stack(dets, axis=0)
